# baseline (speedup 1.0000x reference)
import functools

import jax
import jax.numpy as jnp
from jax import lax
from jax.experimental import pallas as pl
from jax.experimental.pallas import tpu as pltpu
from jax.experimental.pallas import tpu_sc as plsc

N = 10000
E = 320000
CH = 128
NPAD = 10240
NC = 2
NS = 16
NW = NC * NS
EPW = E // NW
K = 125
HALF = NPAD // 2
HPAD = HALF + 128
DUMP = HALF
CAP = 6250
CAPC = CAP // K
RPH = HALF // NS

_sc_mesh = plsc.VectorSubcoreMesh(
    core_axis_name="c", subcore_axis_name="s", num_cores=NC, num_subcores=NS
)
_sc_params = pltpu.CompilerParams(needs_layout_passes=False)


@functools.partial(
    pl.kernel,
    out_type=[
        jax.ShapeDtypeStruct((NW, NPAD), jnp.float32),
        jax.ShapeDtypeStruct((NW, 2, 2, CAP), jnp.int32),
        jax.ShapeDtypeStruct((NW, 16), jnp.int32),
    ],
    mesh=_sc_mesh,
    compiler_params=_sc_params,
    scratch_types=[
        pltpu.VMEM((2, EPW), jnp.int32),
        pltpu.VMEM((NPAD,), jnp.float32),
        pltpu.VMEM((CAP,), jnp.int32),
        pltpu.VMEM((CAP,), jnp.int32),
        pltpu.VMEM((CAP,), jnp.int32),
        pltpu.VMEM((CAP,), jnp.int32),
        pltpu.VMEM((16,), jnp.int32),
    ],
)
def _deg_kernel(
    sd_hbm, hist_hbm, plist_hbm, cnt_hbm,
    sd_v, hist_v, asrc_v, adst_v, bsrc_v, bdst_v, cnt_v,
):
    wid = lax.axis_index("c") * NS + lax.axis_index("s")
    pltpu.sync_copy(sd_hbm.at[wid], sd_v)

    zeros16 = jnp.zeros((16,), jnp.float32)

    def zbody(i, carry):
        hist_v[pl.ds(i * 16, 16)] = zeros16
        return carry

    lax.fori_loop(0, NPAD // 16, zbody, 0)

    ones16 = jnp.ones((16,), jnp.float32)

    def hbody(g, carry):
        off_a, off_b = carry
        src16 = sd_v[0, pl.ds(g * 16, 16)]
        dst16 = sd_v[1, pl.ds(g * 16, 16)]
        plsc.addupdate_scatter(hist_v, [dst16], ones16)
        mask = dst16 < HALF
        nmask = jnp.logical_not(mask)
        m32 = mask.astype(jnp.int32)
        nm32 = nmask.astype(jnp.int32)
        pos_a = off_a + plsc.cumsum(m32) - m32
        pos_b = off_b + plsc.cumsum(nm32) - nm32
        plsc.store_scatter(asrc_v, [pos_a], src16, mask=mask)
        plsc.store_scatter(adst_v, [pos_a], dst16, mask=mask)
        rel_b = dst16 - HALF
        plsc.store_scatter(bsrc_v, [pos_b], src16, mask=nmask)
        plsc.store_scatter(bdst_v, [pos_b], rel_b, mask=nmask)
        cnt_a = jnp.sum(m32)
        return off_a + cnt_a, off_b + (16 - cnt_a)

    off_a, off_b = lax.fori_loop(
        0, EPW // 16, hbody, (jnp.int32(0), jnp.int32(0))
    )

    zeros16i = jnp.zeros((16,), jnp.int32)
    dump16 = jnp.full((16,), DUMP, jnp.int32)
    ii16 = jax.lax.iota(jnp.int32, 16)
    for t in range(8):
        plsc.store_scatter(asrc_v, [off_a + 16 * t + ii16], zeros16i)
        plsc.store_scatter(adst_v, [off_a + 16 * t + ii16], dump16)
        plsc.store_scatter(bsrc_v, [off_b + 16 * t + ii16], zeros16i)
        plsc.store_scatter(bdst_v, [off_b + 16 * t + ii16], dump16)

    cc_a = (off_a + K - 1) // K
    cc_b = (off_b + K - 1) // K
    ii = jax.lax.iota(jnp.int32, 16)
    cnt_v[...] = jnp.where(ii == 0, cc_a, jnp.where(ii == 1, cc_b, 0))

    pltpu.sync_copy(hist_v, hist_hbm.at[wid])
    pltpu.sync_copy(asrc_v, plist_hbm.at[wid, 0, 0])
    pltpu.sync_copy(adst_v, plist_hbm.at[wid, 0, 1])
    pltpu.sync_copy(bsrc_v, plist_hbm.at[wid, 1, 0])
    pltpu.sync_copy(bdst_v, plist_hbm.at[wid, 1, 1])
    pltpu.sync_copy(cnt_v, cnt_hbm.at[wid])


def _mm_body(x_ref, w_ref, h_ref, y_ref, dis_ref):
    deg = jnp.sum(h_ref[...], axis=0) + 1.0
    dis = lax.rsqrt(deg)
    z = jnp.dot(x_ref[...], w_ref[...], preferred_element_type=jnp.float32)
    y_ref[...] = z * dis[:, None]
    dis_ref[...] = dis[:, None]


_MM_BLK = 1024
_mm_call = pl.pallas_call(
    _mm_body,
    grid=(NPAD // _MM_BLK,),
    in_specs=[
        pl.BlockSpec((_MM_BLK, CH), lambda i: (i, 0)),
        pl.BlockSpec((CH, CH), lambda i: (0, 0)),
        pl.BlockSpec((NW, _MM_BLK), lambda i: (0, i)),
    ],
    out_specs=[
        pl.BlockSpec((_MM_BLK, CH), lambda i: (i, 0)),
        pl.BlockSpec((_MM_BLK, 1), lambda i: (i, 0)),
    ],
    out_shape=[
        jax.ShapeDtypeStruct((NPAD, CH), jnp.float32),
        jax.ShapeDtypeStruct((NPAD, 1), jnp.float32),
    ],
)


@functools.partial(
    pl.kernel,
    out_type=jax.ShapeDtypeStruct((NC, HALF, CH), jnp.float32),
    mesh=_sc_mesh,
    compiler_params=_sc_params,
    scratch_types=[
        pltpu.VMEM((2, 2 * CAPC, K), jnp.int32),
        pltpu.VMEM((2, K, CH), jnp.float32),
        pltpu.VMEM((2, 16), jnp.int32),
        pltpu.VMEM((80, CH), jnp.float32),
        pltpu.VMEM((RPH,), jnp.float32),
        pltpu.VMEM((CH,), jnp.float32),
        pltpu.VMEM_SHARED((HPAD, CH), jnp.float32),
        pltpu.SemaphoreType.DMA,
    ],
)
def _agg_kernel(
    y_hbm, plist_hbm, cnt_hbm, dis_hbm, b_hbm, out_hbm,
    ld_v, rows_v, cnt_v, dbuf_v, dis_v, b_v, acc, gsem,
):
    core = lax.axis_index("c")
    sub = lax.axis_index("s")
    base = sub * RPH

    pltpu.sync_copy(
        y_hbm.at[pl.ds(core * HALF + base, RPH)], acc.at[pl.ds(base, RPH)]
    )

    @pl.when(sub == 0)
    def _():
        pltpu.sync_copy(
            y_hbm.at[pl.ds(0, HPAD - HALF)], acc.at[pl.ds(HALF, HPAD - HALF)]
        )

    pltpu.sync_copy(cnt_hbm.at[pl.ds(2 * sub, 2)], cnt_v)
    ii = jax.lax.iota(jnp.int32, 16)
    c0 = jnp.minimum(jnp.sum(jnp.where(ii == core, cnt_v[0, pl.ds(0, 16)], 0)), CAPC)
    c1 = jnp.minimum(jnp.sum(jnp.where(ii == core, cnt_v[1, pl.ds(0, 16)], 0)), CAPC)
    pltpu.sync_copy(plist_hbm.at[2 * sub, core], ld_v.at[:, pl.ds(0, CAPC)])
    pltpu.sync_copy(plist_hbm.at[2 * sub + 1, core], ld_v.at[:, pl.ds(c0, CAPC)])
    total = c0 + c1
    plsc.subcore_barrier()

    def body(t, carry):
        tm = t - 1

        @pl.when(t > 0)
        def _():
            pltpu.make_async_copy(
                y_hbm.at[ld_v.at[0, tm]], rows_v.at[tm % 2], gsem
            ).wait()

        @pl.when(t < total)
        def _():
            pltpu.async_copy(y_hbm.at[ld_v.at[0, t]], rows_v.at[t % 2], gsem)

        @pl.when(t > 0)
        def _():
            pltpu.sync_copy(rows_v.at[tm % 2], acc.at[ld_v.at[1, tm]], add=True)

        return carry

    lax.fori_loop(0, total + 1, body, 0)

    plsc.subcore_barrier()

    pltpu.sync_copy(dis_hbm.at[pl.ds(core * HALF + base, RPH)], dis_v)
    pltpu.sync_copy(b_hbm, b_v)

    def drain(q, carry):
        pltpu.sync_copy(acc.at[pl.ds(base + 80 * q, 80)], dbuf_v)

        def row(r, carry2):
            ridx = jnp.zeros((16,), jnp.int32) + (80 * q + r)
            d = plsc.load_gather(dis_v, [ridx])
            for u in range(CH // 16):
                cs = pl.ds(16 * u, 16)
                dbuf_v[r, cs] = dbuf_v[r, cs] * d + b_v[cs]
            return carry2

        lax.fori_loop(0, 80, row, 0)
        pltpu.sync_copy(dbuf_v, out_hbm.at[core, pl.ds(base + 80 * q, 80)])
        return carry

    lax.fori_loop(0, RPH // 80, drain, 0)


def kernel(x, edge_index, W, b):
    src = edge_index[0].astype(jnp.int32)
    dst = edge_index[1].astype(jnp.int32)
    sd = jnp.stack([src.reshape(NW, EPW), dst.reshape(NW, EPW)], axis=1)
    hist, plist, counts = _deg_kernel(sd)
    x_pad = jnp.pad(x, ((0, NPAD - N), (0, 0)))
    yp, dis = _mm_call(x_pad, W, hist)
    parts = _agg_kernel(
        yp, plist.reshape(NW, 2, 2, CAPC, K), counts, dis.reshape(NPAD), b
    )
    return jnp.concatenate([parts[0], parts[1, : N - HALF]], axis=0)
